# Initial kernel scaffold; baseline (speedup 1.0000x reference)
import functools

import jax
import jax.numpy as jnp
from jax import lax
from jax.experimental import pallas as pl
from jax.experimental.pallas import tpu as pltpu
from jax.experimental.pallas import tpu_sc as plsc

H = 128
WEXT = 144
NPAD = 10240
ROWBLK = 2048
EPS = 1e-16

NC, NS = 2, 16
NW = NC * NS


def _nt_matmul(a, b):
    return lax.dot_general(a, b, (((1,), (1,)), ((), ())),
                           preferred_element_type=jnp.float32)


def _hext_assemble(xp, ws_ref, as_ref, wd_ref, ad_ref, hext_ref, aldst_ref):
    blk = xp.shape[0]
    h = _nt_matmul(xp, ws_ref[...])
    al_s = jnp.dot(h, as_ref[...].T,
                   preferred_element_type=jnp.float32)
    wv = jnp.dot(ad_ref[...], wd_ref[...],
                 preferred_element_type=jnp.float32)
    al_d = jnp.dot(xp, wv.T, preferred_element_type=jnp.float32)
    hext_ref[:, 0:H] = h
    hext_ref[:, H:H + 1] = jnp.ones((blk, 1), jnp.float32)
    hext_ref[:, H + 1:H + 2] = al_s
    hext_ref[:, H + 2:WEXT] = jnp.zeros((blk, WEXT - H - 2), jnp.float32)
    aldst_ref[...] = al_d


def _layer1_body(x_ref, wlin_ref, blin_ref, ws_ref, as_ref, wd_ref, ad_ref,
                 hext_ref, aldst_ref):
    xp = _nt_matmul(x_ref[...], wlin_ref[...]) + blin_ref[...]
    _hext_assemble(xp, ws_ref, as_ref, wd_ref, ad_ref, hext_ref, aldst_ref)


def _layer2_body(p0_ref, p1_ref, bprev_ref, ws_ref, as_ref, wd_ref, ad_ref,
                 hext_ref, aldst_ref):
    s = p0_ref[...] + p1_ref[...]
    den = s[:, H:H + 1]
    xp = jnp.maximum(s[:, 0:H] / (den + EPS) + bprev_ref[...], 0.0)
    _hext_assemble(xp, ws_ref, as_ref, wd_ref, ad_ref, hext_ref, aldst_ref)


def _final_body(p0_ref, p1_ref, bprev_ref, out_ref):
    s = p0_ref[...] + p1_ref[...]
    den = s[:, H:H + 1]
    out_ref[...] = s[:, 0:H] / (den + EPS) + bprev_ref[...]


def _full_spec(shape):
    nd = len(shape)
    return pl.BlockSpec(shape, lambda i: (0,) * nd)


def _row_spec(cols):
    return pl.BlockSpec((ROWBLK, cols), lambda i: (i, 0))


def _tc_layer1(x, wlin, blin, ws, a_s, wd, a_d):
    d = x.shape[1]
    grid = NPAD // ROWBLK
    return pl.pallas_call(
        _layer1_body,
        grid=(grid,),
        in_specs=[_row_spec(d), _full_spec(wlin.shape), _full_spec((1, H)),
                  _full_spec((H, H)), _full_spec((1, H)),
                  _full_spec((H, H)), _full_spec((1, H))],
        out_specs=[_row_spec(WEXT), _row_spec(1)],
        out_shape=[jax.ShapeDtypeStruct((NPAD, WEXT), jnp.float32),
                   jax.ShapeDtypeStruct((NPAD, 1), jnp.float32)],
    )(x, wlin, blin.reshape(1, H), ws, a_s.reshape(1, H), wd,
      a_d.reshape(1, H))


def _tc_layer2(p, bprev, ws, a_s, wd, a_d):
    grid = NPAD // ROWBLK
    return pl.pallas_call(
        _layer2_body,
        grid=(grid,),
        in_specs=[_row_spec(WEXT), _row_spec(WEXT), _full_spec((1, H)),
                  _full_spec((H, H)), _full_spec((1, H)),
                  _full_spec((H, H)), _full_spec((1, H))],
        out_specs=[_row_spec(WEXT), _row_spec(1)],
        out_shape=[jax.ShapeDtypeStruct((NPAD, WEXT), jnp.float32),
                   jax.ShapeDtypeStruct((NPAD, 1), jnp.float32)],
    )(p[0], p[1], bprev.reshape(1, H), ws, a_s.reshape(1, H), wd,
      a_d.reshape(1, H))


def _tc_final(p, bprev):
    grid = NPAD // ROWBLK
    return pl.pallas_call(
        _final_body,
        grid=(grid,),
        in_specs=[_row_spec(WEXT), _row_spec(WEXT), _full_spec((1, H))],
        out_specs=_row_spec(H),
        out_shape=jax.ShapeDtypeStruct((NPAD, H), jnp.float32),
    )(p[0], p[1], bprev.reshape(1, H))


ECHUNK = 80
EGROUPS = ECHUNK // 16

_MESH = plsc.VectorSubcoreMesh(core_axis_name="c", subcore_axis_name="s")


def _conv_sc_body(n_edges_per_w, hext, aldst, src_idx, dst_idx, out,
                  idx_s, idx_d, rows, aldst_v, acc, sem):
    c = lax.axis_index("c")
    s = lax.axis_index("s")
    w = c * NS + s
    nchunks = n_edges_per_w // ECHUNK
    slab = NPAD // NS

    iota = lax.iota(jnp.int32, 16)

    @pl.loop(0, ECHUNK)
    def _zero_rows(r):
        for j in range(WEXT // 16):
            rows[r, pl.ds(j * 16, 16)] = jnp.zeros((16,), jnp.float32)

    @pl.loop(0, slab // ECHUNK)
    def _zero_acc(i):
        pltpu.sync_copy(rows, acc.at[pl.ds(s * slab + i * ECHUNK, ECHUNK)])

    pltpu.sync_copy(aldst, aldst_v)
    plsc.subcore_barrier()

    ebase = w * n_edges_per_w
    rg = [g * 16 + iota for g in range(EGROUPS)]
    c_alsrc = jnp.full((16,), H + 1, jnp.int32)

    @pl.loop(0, nchunks)
    def _chunk(i):
        base = ebase + i * ECHUNK
        pltpu.sync_copy(src_idx.at[pl.ds(base, ECHUNK)], idx_s)
        pltpu.sync_copy(dst_idx.at[pl.ds(base, ECHUNK)], idx_d)
        pltpu.async_copy(hext.at[idx_s], rows, sem).wait()
        exs = []
        for g in range(EGROUPS):
            al_s = plsc.load_gather(rows, [rg[g], c_alsrc])
            dg = idx_d[pl.ds(g * 16, 16)]
            al_d = plsc.load_gather(aldst_v, [dg])
            e = al_s + al_d
            e = jnp.maximum(e, 0.2 * e)
            exs.append(jnp.exp(e))

        @pl.loop(0, H + 1)
        def _scale(col):
            cc = jnp.full((16,), col, jnp.int32)
            for g in range(EGROUPS):
                v = plsc.load_gather(rows, [rg[g], cc])
                plsc.store_scatter(rows, [rg[g], cc], v * exs[g])

        pltpu.sync_copy(rows, acc.at[idx_d], add=True)

    plsc.subcore_barrier()

    @pl.loop(0, slab // ECHUNK)
    def _writeout(i):
        off = s * slab + i * ECHUNK
        pltpu.sync_copy(acc.at[pl.ds(off, ECHUNK)], rows)
        pltpu.sync_copy(rows, out.at[c, pl.ds(off, ECHUNK)])


def _sc_conv(hext, aldst, src_idx, dst_idx, n_edges):
    n_edges_per_w = n_edges // NW
    kern = pl.kernel(
        functools.partial(_conv_sc_body, n_edges_per_w),
        out_type=jax.ShapeDtypeStruct((NC, NPAD, WEXT), jnp.float32),
        mesh=_MESH,
        scratch_types=[
            pltpu.VMEM((ECHUNK,), jnp.int32),
            pltpu.VMEM((ECHUNK,), jnp.int32),
            pltpu.VMEM((ECHUNK, WEXT), jnp.float32),
            pltpu.VMEM((NPAD,), jnp.float32),
            pltpu.VMEM_SHARED((NPAD, WEXT), jnp.float32),
            pltpu.SemaphoreType.DMA,
        ],
    )
    return kern(hext, aldst, src_idx, dst_idx)


ELCHUNK = 112
ELGROUPS = ELCHUNK // 16


def _classify_sc_body(n_el_per_w, d2, t2, ei0, ei1, out,
                      idx_a, idx_b, rows_a, rows_b, pred, sem_a, sem_b):
    c = lax.axis_index("c")
    s = lax.axis_index("s")
    w = c * NS + s
    nchunks = n_el_per_w // ELCHUNK
    iota = lax.iota(jnp.int32, 16)
    rg = [g * 16 + iota for g in range(ELGROUPS)]
    ebase = w * n_el_per_w

    @pl.loop(0, nchunks)
    def _chunk(i):
        base = ebase + i * ELCHUNK
        pltpu.sync_copy(ei0.at[pl.ds(base, ELCHUNK)], idx_a)
        pltpu.sync_copy(ei1.at[pl.ds(base, ELCHUNK)], idx_b)
        cp_a = pltpu.async_copy(d2.at[idx_a], rows_a, sem_a)
        cp_b = pltpu.async_copy(t2.at[idx_b], rows_b, sem_b)
        cp_a.wait()
        cp_b.wait()
        for g in range(ELGROUPS):
            acc0 = jnp.zeros((16,), jnp.float32)

            @pl.loop(0, H, init_carry=acc0)
            def _dot(col, acc):
                cc = jnp.full((16,), col, jnp.int32)
                va = plsc.load_gather(rows_a, [rg[g], cc])
                vb = plsc.load_gather(rows_b, [rg[g], cc])
                return acc + va * vb

            pred[pl.ds(g * 16, 16)] = _dot

        pltpu.sync_copy(pred, out.at[pl.ds(base, ELCHUNK)])


def _sc_classify(d2, t2, ei0, ei1, n_el):
    n_el_per_w = n_el // NW
    kern = pl.kernel(
        functools.partial(_classify_sc_body, n_el_per_w),
        out_type=jax.ShapeDtypeStruct((n_el,), jnp.float32),
        mesh=_MESH,
        scratch_types=[
            pltpu.VMEM((ELCHUNK,), jnp.int32),
            pltpu.VMEM((ELCHUNK,), jnp.int32),
            pltpu.VMEM((ELCHUNK, H), jnp.float32),
            pltpu.VMEM((ELCHUNK, H), jnp.float32),
            pltpu.VMEM((ELCHUNK,), jnp.float32),
            pltpu.SemaphoreType.DMA,
            pltpu.SemaphoreType.DMA,
        ],
    )
    return kern(d2, t2, ei0, ei1)


def kernel(x_dataset, x_task, W_d_lin, b_d_lin, W_t_lin, b_t_lin,
           c1dt_Ws, c1dt_Wd, c1dt_as, c1dt_ad, c1dt_b,
           c1td_Ws, c1td_Wd, c1td_as, c1td_ad, c1td_b,
           c2dt_Ws, c2dt_Wd, c2dt_as, c2dt_ad, c2dt_b,
           c2td_Ws, c2td_Wd, c2td_as, c2td_ad, c2td_b,
           edge_index_dt, edge_index_td, edge_label_index):
    nd = x_dataset.shape[0]
    E = edge_index_dt.shape[1]
    EL = edge_label_index.shape[1]

    xd_p = jnp.pad(x_dataset, ((0, NPAD - nd), (0, 0)))
    xt_p = jnp.pad(x_task, ((0, NPAD - x_task.shape[0]), (0, 0)))

    hext_d1, aldst_td1 = _tc_layer1(xd_p, W_d_lin, b_d_lin,
                                    c1dt_Ws, c1dt_as, c1td_Wd, c1td_ad)
    hext_t1, aldst_dt1 = _tc_layer1(xt_p, W_t_lin, b_t_lin,
                                    c1td_Ws, c1td_as, c1dt_Wd, c1dt_ad)

    src_dt = edge_index_dt[0].astype(jnp.int32)
    dst_dt = edge_index_dt[1].astype(jnp.int32)
    src_td = edge_index_td[0].astype(jnp.int32)
    dst_td = edge_index_td[1].astype(jnp.int32)

    p_t1 = _sc_conv(hext_d1, aldst_dt1[:, 0], src_dt, dst_dt, E)
    p_d1 = _sc_conv(hext_t1, aldst_td1[:, 0], src_td, dst_td, E)

    hext_d2, aldst_td2 = _tc_layer2(p_d1, c1td_b,
                                    c2dt_Ws, c2dt_as, c2td_Wd, c2td_ad)
    hext_t2, aldst_dt2 = _tc_layer2(p_t1, c1dt_b,
                                    c2td_Ws, c2td_as, c2dt_Wd, c2dt_ad)

    p_t2 = _sc_conv(hext_d2, aldst_dt2[:, 0], src_dt, dst_dt, E)
    p_d2 = _sc_conv(hext_t2, aldst_td2[:, 0], src_td, dst_td, E)

    d2 = _tc_final(p_d2, c2td_b)
    t2 = _tc_final(p_t2, c2dt_b)

    el_pad = -(-EL // (NW * ELCHUNK)) * (NW * ELCHUNK)
    ei0 = jnp.pad(edge_label_index[0].astype(jnp.int32), (0, el_pad - EL))
    ei1 = jnp.pad(edge_label_index[1].astype(jnp.int32), (0, el_pad - EL))
    pred = _sc_classify(d2, t2, ei0, ei1, el_pad)
    return pred[:EL]


# speedup vs baseline: 8.1859x; 8.1859x over previous
import functools

import jax
import jax.numpy as jnp
from jax import lax
from jax.experimental import pallas as pl
from jax.experimental.pallas import tpu as pltpu
from jax.experimental.pallas import tpu_sc as plsc

H = 128
WEXT = 144
NPAD = 10240
ROWBLK = 2048
EPS = 1e-16

NC, NS = 2, 16
NW = NC * NS


def _nt_matmul(a, b):
    return lax.dot_general(a, b, (((1,), (1,)), ((), ())),
                           preferred_element_type=jnp.float32)


def _hext_assemble(xp, ws_ref, as_ref, wd_ref, ad_ref, hext_ref, aldst_ref):
    blk = xp.shape[0]
    h = _nt_matmul(xp, ws_ref[...])
    al_s = jnp.dot(h, as_ref[...].T,
                   preferred_element_type=jnp.float32)
    wv = jnp.dot(ad_ref[...], wd_ref[...],
                 preferred_element_type=jnp.float32)
    al_d = jnp.dot(xp, wv.T, preferred_element_type=jnp.float32)
    hext_ref[:, 0:H] = h
    hext_ref[:, H:H + 1] = jnp.ones((blk, 1), jnp.float32)
    hext_ref[:, H + 1:H + 2] = al_s
    hext_ref[:, H + 2:WEXT] = jnp.zeros((blk, WEXT - H - 2), jnp.float32)
    aldst_ref[...] = al_d


def _layer1_body(x_ref, wlin_ref, blin_ref, ws_ref, as_ref, wd_ref, ad_ref,
                 hext_ref, aldst_ref):
    xp = _nt_matmul(x_ref[...], wlin_ref[...]) + blin_ref[...]
    _hext_assemble(xp, ws_ref, as_ref, wd_ref, ad_ref, hext_ref, aldst_ref)


def _layer2_body(p0_ref, p1_ref, bprev_ref, ws_ref, as_ref, wd_ref, ad_ref,
                 hext_ref, aldst_ref):
    s = p0_ref[...] + p1_ref[...]
    den = s[:, H:H + 1]
    xp = jnp.maximum(s[:, 0:H] / (den + EPS) + bprev_ref[...], 0.0)
    _hext_assemble(xp, ws_ref, as_ref, wd_ref, ad_ref, hext_ref, aldst_ref)


def _final_body(p0_ref, p1_ref, bprev_ref, out_ref):
    s = p0_ref[...] + p1_ref[...]
    den = s[:, H:H + 1]
    out_ref[...] = s[:, 0:H] / (den + EPS) + bprev_ref[...]


def _full_spec(shape):
    nd = len(shape)
    return pl.BlockSpec(shape, lambda i: (0,) * nd)


def _row_spec(cols):
    return pl.BlockSpec((ROWBLK, cols), lambda i: (i, 0))


def _tc_layer1(x, wlin, blin, ws, a_s, wd, a_d):
    d = x.shape[1]
    grid = NPAD // ROWBLK
    return pl.pallas_call(
        _layer1_body,
        grid=(grid,),
        in_specs=[_row_spec(d), _full_spec(wlin.shape), _full_spec((1, H)),
                  _full_spec((H, H)), _full_spec((1, H)),
                  _full_spec((H, H)), _full_spec((1, H))],
        out_specs=[_row_spec(WEXT), _row_spec(1)],
        out_shape=[jax.ShapeDtypeStruct((NPAD, WEXT), jnp.float32),
                   jax.ShapeDtypeStruct((NPAD, 1), jnp.float32)],
    )(x, wlin, blin.reshape(1, H), ws, a_s.reshape(1, H), wd,
      a_d.reshape(1, H))


def _tc_layer2(p, bprev, ws, a_s, wd, a_d):
    grid = NPAD // ROWBLK
    return pl.pallas_call(
        _layer2_body,
        grid=(grid,),
        in_specs=[_row_spec(WEXT), _row_spec(WEXT), _full_spec((1, H)),
                  _full_spec((H, H)), _full_spec((1, H)),
                  _full_spec((H, H)), _full_spec((1, H))],
        out_specs=[_row_spec(WEXT), _row_spec(1)],
        out_shape=[jax.ShapeDtypeStruct((NPAD, WEXT), jnp.float32),
                   jax.ShapeDtypeStruct((NPAD, 1), jnp.float32)],
    )(p[0], p[1], bprev.reshape(1, H), ws, a_s.reshape(1, H), wd,
      a_d.reshape(1, H))


def _tc_final(p, bprev):
    grid = NPAD // ROWBLK
    return pl.pallas_call(
        _final_body,
        grid=(grid,),
        in_specs=[_row_spec(WEXT), _row_spec(WEXT), _full_spec((1, H))],
        out_specs=_row_spec(H),
        out_shape=jax.ShapeDtypeStruct((NPAD, H), jnp.float32),
    )(p[0], p[1], bprev.reshape(1, H))


ECHUNK = 80
EGROUPS = ECHUNK // 16

_MESH = plsc.VectorSubcoreMesh(core_axis_name="c", subcore_axis_name="s")


def _conv_sc_body(n_edges_per_w, hext, aldst, src_idx, dst_idx, out,
                  idx_s, idx_d, rows, aldst_v, acc, sem):
    c = lax.axis_index("c")
    s = lax.axis_index("s")
    w = c * NS + s
    nchunks = n_edges_per_w // ECHUNK
    slab = NPAD // NS

    iota = lax.iota(jnp.int32, 16)

    @pl.loop(0, ECHUNK)
    def _zero_rows(r):
        for j in range(WEXT // 16):
            rows[r, pl.ds(j * 16, 16)] = jnp.zeros((16,), jnp.float32)

    @pl.loop(0, slab // ECHUNK)
    def _zero_acc(i):
        pltpu.sync_copy(rows, acc.at[pl.ds(s * slab + i * ECHUNK, ECHUNK)])

    pltpu.sync_copy(aldst, aldst_v)
    plsc.subcore_barrier()

    ebase = w * n_edges_per_w
    rg = [g * 16 + iota for g in range(EGROUPS)]
    c_alsrc = jnp.full((16,), H + 1, jnp.int32)

    @pl.loop(0, nchunks)
    def _chunk(i):
        base = ebase + i * ECHUNK
        pltpu.sync_copy(src_idx.at[pl.ds(base, ECHUNK)], idx_s)
        pltpu.sync_copy(dst_idx.at[pl.ds(base, ECHUNK)], idx_d)
        pltpu.async_copy(hext.at[idx_s], rows, sem).wait()
        exs = []
        for g in range(EGROUPS):
            al_s = plsc.load_gather(rows, [rg[g], c_alsrc])
            dg = idx_d[pl.ds(g * 16, 16)]
            al_d = plsc.load_gather(aldst_v, [dg])
            e = al_s + al_d
            e = jnp.maximum(e, 0.2 * e)
            exs.append(jnp.exp(e))

        @pl.loop(0, H + 1)
        def _scale(col):
            cc = jnp.full((16,), col, jnp.int32)
            for g in range(EGROUPS):
                v = plsc.load_gather(rows, [rg[g], cc])
                plsc.store_scatter(rows, [rg[g], cc], v * exs[g])

        pltpu.sync_copy(rows, acc.at[idx_d], add=True)

    plsc.subcore_barrier()

    @pl.loop(0, slab // ECHUNK)
    def _writeout(i):
        off = s * slab + i * ECHUNK
        pltpu.sync_copy(acc.at[pl.ds(off, ECHUNK)], rows)
        pltpu.sync_copy(rows, out.at[c, pl.ds(off, ECHUNK)])


def _sc_conv(hext, aldst, src_idx, dst_idx, n_edges):
    n_edges_per_w = n_edges // NW
    kern = pl.kernel(
        functools.partial(_conv_sc_body, n_edges_per_w),
        out_type=jax.ShapeDtypeStruct((NC, NPAD, WEXT), jnp.float32),
        mesh=_MESH,
        compiler_params=pltpu.CompilerParams(use_tc_tiling_on_sc=False, needs_layout_passes=False),
        scratch_types=[
            pltpu.VMEM((ECHUNK,), jnp.int32),
            pltpu.VMEM((ECHUNK,), jnp.int32),
            pltpu.VMEM((ECHUNK, WEXT), jnp.float32),
            pltpu.VMEM((NPAD,), jnp.float32),
            pltpu.VMEM_SHARED((NPAD, WEXT), jnp.float32),
            pltpu.SemaphoreType.DMA,
        ],
    )
    return kern(hext, aldst, src_idx, dst_idx)


ELCHUNK = 112
ELGROUPS = ELCHUNK // 16


def _classify_sc_body(n_el_per_w, d2, t2, ei0, ei1, out,
                      idx_a, idx_b, rows_a, rows_b, pred, sem_a, sem_b):
    c = lax.axis_index("c")
    s = lax.axis_index("s")
    w = c * NS + s
    nchunks = n_el_per_w // ELCHUNK
    iota = lax.iota(jnp.int32, 16)
    rg = [g * 16 + iota for g in range(ELGROUPS)]
    ebase = w * n_el_per_w

    @pl.loop(0, nchunks)
    def _chunk(i):
        base = ebase + i * ELCHUNK
        pltpu.sync_copy(ei0.at[pl.ds(base, ELCHUNK)], idx_a)
        pltpu.sync_copy(ei1.at[pl.ds(base, ELCHUNK)], idx_b)
        cp_a = pltpu.async_copy(d2.at[idx_a], rows_a, sem_a)
        cp_b = pltpu.async_copy(t2.at[idx_b], rows_b, sem_b)
        cp_a.wait()
        cp_b.wait()
        for g in range(ELGROUPS):
            acc0 = jnp.zeros((16,), jnp.float32)

            @pl.loop(0, H, init_carry=acc0)
            def _dot(col, acc):
                cc = jnp.full((16,), col, jnp.int32)
                va = plsc.load_gather(rows_a, [rg[g], cc])
                vb = plsc.load_gather(rows_b, [rg[g], cc])
                return acc + va * vb

            pred[pl.ds(g * 16, 16)] = _dot

        pltpu.sync_copy(pred, out.at[pl.ds(base, ELCHUNK)])


def _sc_classify(d2, t2, ei0, ei1, n_el):
    n_el_per_w = n_el // NW
    kern = pl.kernel(
        functools.partial(_classify_sc_body, n_el_per_w),
        out_type=jax.ShapeDtypeStruct((n_el,), jnp.float32),
        mesh=_MESH,
        compiler_params=pltpu.CompilerParams(use_tc_tiling_on_sc=False, needs_layout_passes=False),
        scratch_types=[
            pltpu.VMEM((ELCHUNK,), jnp.int32),
            pltpu.VMEM((ELCHUNK,), jnp.int32),
            pltpu.VMEM((ELCHUNK, H), jnp.float32),
            pltpu.VMEM((ELCHUNK, H), jnp.float32),
            pltpu.VMEM((ELCHUNK,), jnp.float32),
            pltpu.SemaphoreType.DMA,
            pltpu.SemaphoreType.DMA,
        ],
    )
    return kern(d2, t2, ei0, ei1)


def kernel(x_dataset, x_task, W_d_lin, b_d_lin, W_t_lin, b_t_lin,
           c1dt_Ws, c1dt_Wd, c1dt_as, c1dt_ad, c1dt_b,
           c1td_Ws, c1td_Wd, c1td_as, c1td_ad, c1td_b,
           c2dt_Ws, c2dt_Wd, c2dt_as, c2dt_ad, c2dt_b,
           c2td_Ws, c2td_Wd, c2td_as, c2td_ad, c2td_b,
           edge_index_dt, edge_index_td, edge_label_index):
    nd = x_dataset.shape[0]
    E = edge_index_dt.shape[1]
    EL = edge_label_index.shape[1]

    xd_p = jnp.pad(x_dataset, ((0, NPAD - nd), (0, 0)))
    xt_p = jnp.pad(x_task, ((0, NPAD - x_task.shape[0]), (0, 0)))

    hext_d1, aldst_td1 = _tc_layer1(xd_p, W_d_lin, b_d_lin,
                                    c1dt_Ws, c1dt_as, c1td_Wd, c1td_ad)
    hext_t1, aldst_dt1 = _tc_layer1(xt_p, W_t_lin, b_t_lin,
                                    c1td_Ws, c1td_as, c1dt_Wd, c1dt_ad)

    src_dt = edge_index_dt[0].astype(jnp.int32)
    dst_dt = edge_index_dt[1].astype(jnp.int32)
    src_td = edge_index_td[0].astype(jnp.int32)
    dst_td = edge_index_td[1].astype(jnp.int32)

    p_t1 = _sc_conv(hext_d1, aldst_dt1[:, 0], src_dt, dst_dt, E)
    p_d1 = _sc_conv(hext_t1, aldst_td1[:, 0], src_td, dst_td, E)

    hext_d2, aldst_td2 = _tc_layer2(p_d1, c1td_b,
                                    c2dt_Ws, c2dt_as, c2td_Wd, c2td_ad)
    hext_t2, aldst_dt2 = _tc_layer2(p_t1, c1dt_b,
                                    c2td_Ws, c2td_as, c2dt_Wd, c2dt_ad)

    p_t2 = _sc_conv(hext_d2, aldst_dt2[:, 0], src_dt, dst_dt, E)
    p_d2 = _sc_conv(hext_t2, aldst_td2[:, 0], src_td, dst_td, E)

    d2 = _tc_final(p_d2, c2td_b)
    t2 = _tc_final(p_t2, c2dt_b)

    el_pad = -(-EL // (NW * ELCHUNK)) * (NW * ELCHUNK)
    ei0 = jnp.pad(edge_label_index[0].astype(jnp.int32), (0, el_pad - EL))
    ei1 = jnp.pad(edge_label_index[1].astype(jnp.int32), (0, el_pad - EL))
    pred = _sc_classify(d2, t2, ei0, ei1, el_pad)
    return pred[:EL]


# speedup vs baseline: 13.9140x; 1.6998x over previous
import functools

import jax
import jax.numpy as jnp
from jax import lax
from jax.experimental import pallas as pl
from jax.experimental.pallas import tpu as pltpu
from jax.experimental.pallas import tpu_sc as plsc

H = 128
WEXT = 136
NPAD = 10240
ROWBLK = 2048
EPS = 1e-16

NC, NS = 2, 16
NW = NC * NS


def _nt_matmul(a, b):
    return lax.dot_general(a, b, (((1,), (1,)), ((), ())),
                           preferred_element_type=jnp.float32)


def _hext_assemble(xp, ws_ref, as_ref, wd_ref, ad_ref, hext_ref, aldst_ref):
    blk = xp.shape[0]
    h = _nt_matmul(xp, ws_ref[...])
    al_s = jnp.dot(h, as_ref[...].T,
                   preferred_element_type=jnp.float32)
    wv = jnp.dot(ad_ref[...], wd_ref[...],
                 preferred_element_type=jnp.float32)
    al_d = jnp.dot(xp, wv.T, preferred_element_type=jnp.float32)
    hext_ref[:, 0:H] = h
    hext_ref[:, H:H + 1] = jnp.ones((blk, 1), jnp.float32)
    hext_ref[:, H + 1:H + 2] = al_s
    hext_ref[:, H + 2:WEXT] = jnp.zeros((blk, WEXT - H - 2), jnp.float32)
    aldst_ref[...] = al_d


def _layer1_body(x_ref, wlin_ref, blin_ref, ws_ref, as_ref, wd_ref, ad_ref,
                 hext_ref, aldst_ref):
    xp = _nt_matmul(x_ref[...], wlin_ref[...]) + blin_ref[...]
    _hext_assemble(xp, ws_ref, as_ref, wd_ref, ad_ref, hext_ref, aldst_ref)


def _layer2_body(p0_ref, p1_ref, bprev_ref, ws_ref, as_ref, wd_ref, ad_ref,
                 hext_ref, aldst_ref):
    s = p0_ref[...] + p1_ref[...]
    den = s[:, H:H + 1]
    xp = jnp.maximum(s[:, 0:H] / (den + EPS) + bprev_ref[...], 0.0)
    _hext_assemble(xp, ws_ref, as_ref, wd_ref, ad_ref, hext_ref, aldst_ref)


def _final_body(p0_ref, p1_ref, bprev_ref, out_ref):
    s = p0_ref[...] + p1_ref[...]
    den = s[:, H:H + 1]
    out_ref[...] = s[:, 0:H] / (den + EPS) + bprev_ref[...]


def _full_spec(shape):
    nd = len(shape)
    return pl.BlockSpec(shape, lambda i: (0,) * nd)


def _row_spec(cols):
    return pl.BlockSpec((ROWBLK, cols), lambda i: (i, 0))


def _tc_layer1(x, wlin, blin, ws, a_s, wd, a_d):
    d = x.shape[1]
    grid = NPAD // ROWBLK
    return pl.pallas_call(
        _layer1_body,
        grid=(grid,),
        in_specs=[_row_spec(d), _full_spec(wlin.shape), _full_spec((1, H)),
                  _full_spec((H, H)), _full_spec((1, H)),
                  _full_spec((H, H)), _full_spec((1, H))],
        out_specs=[_row_spec(WEXT), _row_spec(1)],
        out_shape=[jax.ShapeDtypeStruct((NPAD, WEXT), jnp.float32),
                   jax.ShapeDtypeStruct((NPAD, 1), jnp.float32)],
    )(x, wlin, blin.reshape(1, H), ws, a_s.reshape(1, H), wd,
      a_d.reshape(1, H))


def _tc_layer2(p, bprev, ws, a_s, wd, a_d):
    grid = NPAD // ROWBLK
    return pl.pallas_call(
        _layer2_body,
        grid=(grid,),
        in_specs=[_row_spec(WEXT), _row_spec(WEXT), _full_spec((1, H)),
                  _full_spec((H, H)), _full_spec((1, H)),
                  _full_spec((H, H)), _full_spec((1, H))],
        out_specs=[_row_spec(WEXT), _row_spec(1)],
        out_shape=[jax.ShapeDtypeStruct((NPAD, WEXT), jnp.float32),
                   jax.ShapeDtypeStruct((NPAD, 1), jnp.float32)],
    )(p[0], p[1], bprev.reshape(1, H), ws, a_s.reshape(1, H), wd,
      a_d.reshape(1, H))


def _tc_final(p, bprev):
    grid = NPAD // ROWBLK
    return pl.pallas_call(
        _final_body,
        grid=(grid,),
        in_specs=[_row_spec(WEXT), _row_spec(WEXT), _full_spec((1, H))],
        out_specs=_row_spec(H),
        out_shape=jax.ShapeDtypeStruct((NPAD, H), jnp.float32),
    )(p[0], p[1], bprev.reshape(1, H))


ECHUNK = 48
EGROUPS = ECHUNK // 16
NBUF = 4
EPW = 10176
NCH = EPW // ECHUNK

_MESH = plsc.VectorSubcoreMesh(core_axis_name="c", subcore_axis_name="s")


def _conv_sc_body(hext, aldst, src_idx, dst_idx, out,
                  is0, is1, is2, is3, id0, id1, id2, id3,
                  r0, r1, r2, r3, aldst_v, acc,
                  gi0, gi1, gi2, gi3, g0, g1, g2, g3, s0, s1, s2, s3):
    c = lax.axis_index("c")
    s = lax.axis_index("s")
    w = c * NS + s
    slab = NPAD // NS
    idxs = [is0, is1, is2, is3]
    idxd = [id0, id1, id2, id3]
    rows = [r0, r1, r2, r3]
    isem = [gi0, gi1, gi2, gi3]
    gsem = [g0, g1, g2, g3]
    ssem = [s0, s1, s2, s3]
    ebase = w * EPW

    iota = lax.iota(jnp.int32, 16)

    @pl.loop(0, 16)
    def _zero_rows(r):
        for j in range(8):
            r0[r, pl.ds(j * 16, 16)] = jnp.zeros((16,), jnp.float32)
        r0[r, pl.ds(WEXT - 16, 16)] = jnp.zeros((16,), jnp.float32)

    @pl.loop(0, slab // 16)
    def _zero_acc(i):
        pltpu.sync_copy(r0.at[pl.ds(0, 16)],
                        acc.at[pl.ds(s * slab + i * 16, 16)])

    pltpu.sync_copy(aldst, aldst_v)
    plsc.subcore_barrier()

    rg = [g * 16 + iota for g in range(EGROUPS)]
    c_alsrc = jnp.full((16,), H + 1, jnp.int32)

    def start_idx(cc, b):
        pltpu.async_copy(src_idx.at[pl.ds(ebase + cc * ECHUNK, ECHUNK)],
                         idxs[b], isem[b])
        pltpu.async_copy(dst_idx.at[pl.ds(ebase + cc * ECHUNK, ECHUNK)],
                         idxd[b], isem[b])

    def wait_idx(b):
        pltpu.make_async_copy(src_idx.at[pl.ds(0, ECHUNK)], idxs[b],
                              isem[b]).wait()
        pltpu.make_async_copy(dst_idx.at[pl.ds(0, ECHUNK)], idxd[b],
                              isem[b]).wait()

    def wait_gather(b):
        pltpu.make_async_copy(hext.at[idxs[b]], rows[b], gsem[b]).wait()

    def wait_scatter(b):
        pltpu.make_async_copy(rows[b], acc.at[idxd[b]], ssem[b]).wait()

    def step(cc, b):
        b3 = (b + 3) % NBUF
        b2 = (b + 2) % NBUF

        @pl.when(cc + 3 < NCH)
        def _idx():
            start_idx(cc + 3, b3)

        @pl.when(cc + 2 < NCH)
        def _gather():
            wait_idx(b2)

            @pl.when(cc >= 2)
            def _():
                wait_scatter(b2)
            pltpu.async_copy(hext.at[idxs[b2]], rows[b2], gsem[b2])

        wait_gather(b)
        exs = []
        for g in range(EGROUPS):
            al_s = plsc.load_gather(rows[b], [rg[g], c_alsrc])
            dg = idxd[b][pl.ds(g * 16, 16)]
            al_d = plsc.load_gather(aldst_v, [dg])
            e = al_s + al_d
            e = jnp.maximum(e, 0.2 * e)
            exs.append(jnp.exp(e))

        @pl.loop(0, H + 1, unroll=3)
        def _scale(col):
            ccol = jnp.full((16,), col, jnp.int32)
            for g in range(EGROUPS):
                v = plsc.load_gather(rows[b], [rg[g], ccol])
                plsc.store_scatter(rows[b], [rg[g], ccol], v * exs[g])

        pltpu.async_copy(rows[b], acc.at[idxd[b]], ssem[b], add=True)

    start_idx(0, 0)
    start_idx(1, 1)
    start_idx(2, 2)
    wait_idx(0)
    pltpu.async_copy(hext.at[idxs[0]], rows[0], gsem[0])
    wait_idx(1)
    pltpu.async_copy(hext.at[idxs[1]], rows[1], gsem[1])

    @pl.loop(0, NCH // NBUF)
    def _quad(i):
        for b in range(NBUF):
            step(i * NBUF + b, b)

    for b in range(NBUF):
        wait_scatter(b)
    plsc.subcore_barrier()

    pltpu.sync_copy(acc.at[pl.ds(s * slab, slab)],
                    out.at[c, pl.ds(s * slab, slab)])


def _sc_conv(hext, aldst, src_idx, dst_idx):
    kern = pl.kernel(
        _conv_sc_body,
        out_type=jax.ShapeDtypeStruct((NC, NPAD, WEXT), jnp.float32),
        mesh=_MESH,
        compiler_params=pltpu.CompilerParams(use_tc_tiling_on_sc=False, needs_layout_passes=False),
        scratch_types=(
            [pltpu.VMEM((ECHUNK,), jnp.int32) for _ in range(8)] +
            [pltpu.VMEM((ECHUNK, WEXT), jnp.float32) for _ in range(4)] +
            [pltpu.VMEM((NPAD,), jnp.float32),
             pltpu.VMEM_SHARED((NPAD, WEXT), jnp.float32)] +
            [pltpu.SemaphoreType.DMA for _ in range(12)]
        ),
    )
    return kern(hext, aldst, src_idx, dst_idx)


ELCHUNK = 112
ELGROUPS = ELCHUNK // 16


def _classify_sc_body(nch, d2, t2, ei02, ei12, out,
                      idxa2, idxb2, ra0, ra1, rb0, rb1, pred,
                      ga0, ga1, gb0, gb1):
    c = lax.axis_index("c")
    s = lax.axis_index("s")
    w = c * NS + s
    iota = lax.iota(jnp.int32, 16)
    rg = [g * 16 + iota for g in range(ELGROUPS)]
    ras = [ra0, ra1]
    rbs = [rb0, rb1]
    gas = [ga0, ga1]
    gbs = [gb0, gb1]

    pltpu.sync_copy(ei02.at[pl.ds(w * nch, nch)], idxa2)
    pltpu.sync_copy(ei12.at[pl.ds(w * nch, nch)], idxb2)

    pltpu.async_copy(d2.at[idxa2.at[0]], ras[0], gas[0])
    pltpu.async_copy(t2.at[idxb2.at[0]], rbs[0], gbs[0])

    def step(cc, b):
        nb = 1 - b

        @pl.when(cc + 1 < nch)
        def _refill():
            pltpu.async_copy(d2.at[idxa2.at[cc + 1]], ras[nb], gas[nb])
            pltpu.async_copy(t2.at[idxb2.at[cc + 1]], rbs[nb], gbs[nb])

        pltpu.make_async_copy(d2.at[idxa2.at[0]], ras[b], gas[b]).wait()
        pltpu.make_async_copy(t2.at[idxb2.at[0]], rbs[b], gbs[b]).wait()
        for g in range(ELGROUPS):
            acc0 = jnp.zeros((16,), jnp.float32)

            @pl.loop(0, H, init_carry=acc0, unroll=4)
            def _dot(col, acc):
                ccol = jnp.full((16,), col, jnp.int32)
                va = plsc.load_gather(ras[b], [rg[g], ccol])
                vb = plsc.load_gather(rbs[b], [rg[g], ccol])
                return acc + va * vb

            pred[pl.ds(g * 16, 16)] = _dot

        pltpu.sync_copy(pred, out.at[pl.ds(w * nch * ELCHUNK + cc * ELCHUNK,
                                           ELCHUNK)])

    @pl.loop(0, nch // 2)
    def _pair(i):
        for b in range(2):
            step(i * 2 + b, b)


def _sc_classify(d2, t2, ei02, ei12, n_el, nch):
    kern = pl.kernel(
        functools.partial(_classify_sc_body, nch),
        out_type=jax.ShapeDtypeStruct((n_el,), jnp.float32),
        mesh=_MESH,
        compiler_params=pltpu.CompilerParams(use_tc_tiling_on_sc=False, needs_layout_passes=False),
        scratch_types=[
            pltpu.VMEM((nch, ELCHUNK), jnp.int32),
            pltpu.VMEM((nch, ELCHUNK), jnp.int32),
            pltpu.VMEM((ELCHUNK, H), jnp.float32),
            pltpu.VMEM((ELCHUNK, H), jnp.float32),
            pltpu.VMEM((ELCHUNK, H), jnp.float32),
            pltpu.VMEM((ELCHUNK, H), jnp.float32),
            pltpu.VMEM((ELCHUNK,), jnp.float32),
            pltpu.SemaphoreType.DMA,
            pltpu.SemaphoreType.DMA,
            pltpu.SemaphoreType.DMA,
            pltpu.SemaphoreType.DMA,
        ],
    )
    return kern(d2, t2, ei02, ei12)


def kernel(x_dataset, x_task, W_d_lin, b_d_lin, W_t_lin, b_t_lin,
           c1dt_Ws, c1dt_Wd, c1dt_as, c1dt_ad, c1dt_b,
           c1td_Ws, c1td_Wd, c1td_as, c1td_ad, c1td_b,
           c2dt_Ws, c2dt_Wd, c2dt_as, c2dt_ad, c2dt_b,
           c2td_Ws, c2td_Wd, c2td_as, c2td_ad, c2td_b,
           edge_index_dt, edge_index_td, edge_label_index):
    nd = x_dataset.shape[0]
    E = edge_index_dt.shape[1]
    EL = edge_label_index.shape[1]

    xd_p = jnp.pad(x_dataset, ((0, NPAD - nd), (0, 0)))
    xt_p = jnp.pad(x_task, ((0, NPAD - x_task.shape[0]), (0, 0)))

    hext_d1, aldst_td1 = _tc_layer1(xd_p, W_d_lin, b_d_lin,
                                    c1dt_Ws, c1dt_as, c1td_Wd, c1td_ad)
    hext_t1, aldst_dt1 = _tc_layer1(xt_p, W_t_lin, b_t_lin,
                                    c1td_Ws, c1td_as, c1dt_Wd, c1dt_ad)

    e_pad = EPW * NW - E

    def _prep(row, fill):
        return jnp.pad(row.astype(jnp.int32), (0, e_pad), constant_values=fill)

    src_dt = _prep(edge_index_dt[0], 0)
    dst_dt = _prep(edge_index_dt[1], NPAD - 1)
    src_td = _prep(edge_index_td[0], 0)
    dst_td = _prep(edge_index_td[1], NPAD - 1)

    p_t1 = _sc_conv(hext_d1, aldst_dt1[:, 0], src_dt, dst_dt)
    p_d1 = _sc_conv(hext_t1, aldst_td1[:, 0], src_td, dst_td)

    hext_d2, aldst_td2 = _tc_layer2(p_d1, c1td_b,
                                    c2dt_Ws, c2dt_as, c2td_Wd, c2td_ad)
    hext_t2, aldst_dt2 = _tc_layer2(p_t1, c1dt_b,
                                    c2td_Ws, c2td_as, c2dt_Wd, c2dt_ad)

    p_t2 = _sc_conv(hext_d2, aldst_dt2[:, 0], src_dt, dst_dt)
    p_d2 = _sc_conv(hext_t2, aldst_td2[:, 0], src_td, dst_td)

    d2 = _tc_final(p_d2, c2td_b)
    t2 = _tc_final(p_t2, c2dt_b)

    nch_l = 2 * (-(-EL // (NW * ELCHUNK * 2)))
    el_pad = NW * ELCHUNK * nch_l
    ei0 = jnp.pad(edge_label_index[0].astype(jnp.int32),
                  (0, el_pad - EL)).reshape(-1, ELCHUNK)
    ei1 = jnp.pad(edge_label_index[1].astype(jnp.int32),
                  (0, el_pad - EL)).reshape(-1, ELCHUNK)
    pred = _sc_classify(d2, t2, ei0, ei1, el_pad, nch_l)
    return pred[:EL]
